# baseline (speedup 1.0000x reference)
import functools

import jax
import jax.numpy as jnp
from jax import lax
from jax.experimental import pallas as pl
from jax.experimental.pallas import tpu as pltpu
from jax.experimental.pallas import tpu_sc as plsc

C = 26
V = 100000
VP = 100352
DM = 32
L = 20
NC, NS = 2, 16
NW = NC * NS
LANES = 16

SUP = 512
NT = VP // SUP

NBC = 16
LW = 5
NWAVE = L // LW
RW = LW * NBC
IC = RW * C
GS = 104
NG = IC // GS


def _transpose_body(t_ref, o_ref):
    blk = t_ref[0]
    o_ref[...] = jnp.concatenate(
        [jnp.transpose(blk[:, q * 128:(q + 1) * 128], (1, 0))
         for q in range(4)],
        axis=1,
    )


def _lookup_body(batch, x_hbm, tables_hbm, out_hbm, xv, idxv, rows, outv, sem):
    wid = lax.axis_index("s") * NC + lax.axis_index("c")
    b_per_w = batch // NW
    nchunks = b_per_w // NBC

    def chunk(g, carry):
        b0 = wid * b_per_w + g * NBC
        pltpu.sync_copy(x_hbm.at[:, :, pl.ds(b0, NBC)], xv)

        for w in range(NWAVE):
            def mkidx(t, c2):
                c = t // LW
                dl = t - c * LW
                v = xv[c, w * LW + dl, :]
                idxv[pl.ds(t * LANES, LANES)] = (
                    c * VP
                    + ((v >> 9) << 9)
                    + ((v & 127) << 2)
                    + ((v >> 7) & 3)
                )
                return c2
            lax.fori_loop(0, IC // LANES, mkidx, 0)

            cps = [
                pltpu.async_copy(
                    tables_hbm.at[idxv.at[pl.ds(j * GS, GS)]],
                    rows.at[pl.ds(j * GS, GS)],
                    sem,
                )
                for j in range(NG)
            ]
            for cp in cps:
                cp.wait()

            def srow(q, c2):
                dl = q // NBC
                db = q - dl * NBC
                a0 = rows[q, pl.ds(0, LANES)]
                a1 = rows[q, pl.ds(LANES, LANES)]
                for c in range(1, C):
                    a0 = a0 + rows[q + RW * c, pl.ds(0, LANES)]
                    a1 = a1 + rows[q + RW * c, pl.ds(LANES, LANES)]
                outv[db, w * LW + dl, pl.ds(0, LANES)] = a0
                outv[db, w * LW + dl, pl.ds(LANES, LANES)] = a1
                return c2
            lax.fori_loop(0, RW, srow, 0)

        pltpu.sync_copy(outv, out_hbm.at[pl.ds(b0, NBC)])
        return carry

    lax.fori_loop(0, nchunks, chunk, 0)


@jax.jit
def kernel(x, tables):
    B, sl, c = x.shape
    assert c == C and sl == L and tables.shape == (C, V, DM)
    assert B % (NW * NBC) == 0

    xt = jnp.transpose(x, (2, 1, 0))
    t3 = jnp.transpose(tables, (0, 2, 1))

    flat = pl.pallas_call(
        _transpose_body,
        grid=(C, NT),
        in_specs=[
            pl.BlockSpec((1, DM, SUP), lambda ci, vt: (ci, 0, vt)),
        ],
        out_specs=pl.BlockSpec((128, 128), lambda ci, vt: (ci * NT + vt, 0)),
        out_shape=jax.ShapeDtypeStruct((C * VP * DM // 128, 128),
                                       jnp.float32),
    )(t3)
    tables_flat = flat.reshape(C * VP, DM)

    mesh = plsc.VectorSubcoreMesh(core_axis_name="c", subcore_axis_name="s")
    call = pl.kernel(
        functools.partial(_lookup_body, B),
        out_type=jax.ShapeDtypeStruct((B, L, DM), jnp.float32),
        mesh=mesh,
        compiler_params=pltpu.CompilerParams(use_tc_tiling_on_sc=False),
        scratch_types=[
            pltpu.VMEM((C, L, NBC), jnp.int32),
            pltpu.VMEM((IC,), jnp.int32),
            pltpu.VMEM((IC, DM), jnp.float32),
            pltpu.VMEM((NBC, L, DM), jnp.float32),
            pltpu.SemaphoreType.DMA,
        ],
    )
    return call(xt, tables_flat)


# speedup vs baseline: 2.2745x; 2.2745x over previous
import functools

import jax
import jax.numpy as jnp
from jax import lax
from jax.experimental import pallas as pl
from jax.experimental.pallas import tpu as pltpu
from jax.experimental.pallas import tpu_sc as plsc

C = 26
V = 100000
DM = 32
L = 20
NC, NS = 2, 16
NW = NC * NS
LANES = 16

NBC = 16
LW = 5
NWAVE = L // LW
RW = LW * NBC
IC = RW * C
GS = 104
NG = IC // GS


def _body(batch, x_hbm, tables_hbm, out_hbm, xv, idxv, rows, outv, sem):
    wid = lax.axis_index("s") * NC + lax.axis_index("c")
    b_per_w = batch // NW
    nchunks = b_per_w // NBC

    def chunk(g, carry):
        b0 = wid * b_per_w + g * NBC
        pltpu.sync_copy(x_hbm.at[:, :, pl.ds(b0, NBC)], xv)

        for w in range(NWAVE):
            def mkidx(t, c2):
                c = t // LW
                dl = t - c * LW
                idxv[pl.ds(t * LANES, LANES)] = xv[c, w * LW + dl, :] + c * V
                return c2
            lax.fori_loop(0, IC // LANES, mkidx, 0)

            cps = [
                pltpu.async_copy(
                    tables_hbm.at[idxv.at[pl.ds(j * GS, GS)]],
                    rows.at[pl.ds(j * GS, GS)],
                    sem,
                )
                for j in range(NG)
            ]
            for cp in cps:
                cp.wait()

            def srow(q, c2):
                dl = q // NBC
                db = q - dl * NBC
                a0 = rows[q, pl.ds(0, LANES)]
                a1 = rows[q, pl.ds(LANES, LANES)]
                for c in range(1, C):
                    a0 = a0 + rows[q + RW * c, pl.ds(0, LANES)]
                    a1 = a1 + rows[q + RW * c, pl.ds(LANES, LANES)]
                outv[db, w * LW + dl, pl.ds(0, LANES)] = a0
                outv[db, w * LW + dl, pl.ds(LANES, LANES)] = a1
                return c2
            lax.fori_loop(0, RW, srow, 0)

        pltpu.sync_copy(outv, out_hbm.at[pl.ds(b0, NBC)])
        return carry

    lax.fori_loop(0, nchunks, chunk, 0)


@jax.jit
def kernel(x, tables):
    B, sl, c = x.shape
    assert c == C and sl == L and tables.shape == (C, V, DM)
    assert B % (NW * NBC) == 0

    xt = jnp.transpose(x, (2, 1, 0))
    tables_flat = tables.reshape(C * V, DM)

    mesh = plsc.VectorSubcoreMesh(core_axis_name="c", subcore_axis_name="s")
    call = pl.kernel(
        functools.partial(_body, B),
        out_type=jax.ShapeDtypeStruct((B, L, DM), jnp.float32),
        mesh=mesh,
        compiler_params=pltpu.CompilerParams(use_tc_tiling_on_sc=False),
        scratch_types=[
            pltpu.VMEM((C, L, NBC), jnp.int32),
            pltpu.VMEM((IC,), jnp.int32),
            pltpu.VMEM((IC, DM), jnp.float32),
            pltpu.VMEM((NBC, L, DM), jnp.float32),
            pltpu.SemaphoreType.DMA,
        ],
    )
    return call(xt, tables_flat)


# speedup vs baseline: 2.3935x; 1.0523x over previous
import functools

import jax
import jax.numpy as jnp
from jax import lax
from jax.experimental import pallas as pl
from jax.experimental.pallas import tpu as pltpu
from jax.experimental.pallas import tpu_sc as plsc

C = 26
V = 100000
DM = 32
L = 20
NC, NS = 2, 16
NW = NC * NS
LANES = 16

NBC = 16
LW = 2
NWAVE = L // LW
RW = LW * NBC
IC = RW * C
GS = 104
NG = IC // GS


def _body(batch, x_hbm, tables_hbm, out_hbm, xv, idxv, rows, outv, sem0,
          sem1):
    wid = lax.axis_index("s") * NC + lax.axis_index("c")
    b_per_w = batch // NW
    nchunks = b_per_w // NBC
    sems = (sem0, sem1)

    def chunk(g, carry):
        b0 = wid * b_per_w + g * NBC
        pltpu.sync_copy(x_hbm.at[:, :, pl.ds(b0, NBC)], xv)

        def mkidx_wave(w):
            def mkidx(t, c2):
                c = t // LW
                dl = t - c * LW
                idxv[w % 2, pl.ds(t * LANES, LANES)] = (
                    xv[c, w * LW + dl, :] + c * V
                )
                return c2
            lax.fori_loop(0, IC // LANES, mkidx, 0)

        def fire(w):
            return [
                pltpu.async_copy(
                    tables_hbm.at[idxv.at[w % 2, pl.ds(j * GS, GS)]],
                    rows.at[w % 2, pl.ds(j * GS, GS)],
                    sems[w % 2],
                )
                for j in range(NG)
            ]

        mkidx_wave(0)
        cps = fire(0)
        for w in range(NWAVE):
            nxt = None
            if w + 1 < NWAVE:
                mkidx_wave(w + 1)
                nxt = fire(w + 1)
            for cp in cps:
                cp.wait()
            cps = nxt

            def srow(q, c2):
                dl = q // NBC
                db = q - dl * NBC
                a0 = rows[w % 2, q, pl.ds(0, LANES)]
                a1 = rows[w % 2, q, pl.ds(LANES, LANES)]
                for c in range(1, C):
                    a0 = a0 + rows[w % 2, q + RW * c, pl.ds(0, LANES)]
                    a1 = a1 + rows[w % 2, q + RW * c, pl.ds(LANES, LANES)]
                outv[db, w * LW + dl, pl.ds(0, LANES)] = a0
                outv[db, w * LW + dl, pl.ds(LANES, LANES)] = a1
                return c2
            lax.fori_loop(0, RW, srow, 0)

        pltpu.sync_copy(outv, out_hbm.at[pl.ds(b0, NBC)])
        return carry

    lax.fori_loop(0, nchunks, chunk, 0)


@jax.jit
def kernel(x, tables):
    B, sl, c = x.shape
    assert c == C and sl == L and tables.shape == (C, V, DM)
    assert B % (NW * NBC) == 0

    xt = jnp.transpose(x, (2, 1, 0))
    tables_flat = tables.reshape(C * V, DM)

    mesh = plsc.VectorSubcoreMesh(core_axis_name="c", subcore_axis_name="s")
    call = pl.kernel(
        functools.partial(_body, B),
        out_type=jax.ShapeDtypeStruct((B, L, DM), jnp.float32),
        mesh=mesh,
        compiler_params=pltpu.CompilerParams(use_tc_tiling_on_sc=False),
        scratch_types=[
            pltpu.VMEM((C, L, NBC), jnp.int32),
            pltpu.VMEM((2, IC), jnp.int32),
            pltpu.VMEM((2, IC, DM), jnp.float32),
            pltpu.VMEM((NBC, L, DM), jnp.float32),
            pltpu.SemaphoreType.DMA,
            pltpu.SemaphoreType.DMA,
        ],
    )
    return call(xt, tables_flat)
